# Initial kernel scaffold; baseline (speedup 1.0000x reference)
import jax
import jax.numpy as jnp
from jax.experimental import pallas as pl


def kernel(ids, token_type_ids, word_table, pos_table, type_table, ln_gamma, ln_beta):
    raise NotImplementedError("write your pallas kernel here")


# speedup vs baseline: 1.6722x; 1.6722x over previous
import functools

import jax
import jax.numpy as jnp
from jax import lax
from jax.experimental import pallas as pl
from jax.experimental.pallas import tpu as pltpu
from jax.experimental.pallas import tpu_sc as plsc

D = 128
CHUNK = 128


def _sc_gather(ids2d, word_table):
    n_chunks_total, chunk = ids2d.shape
    ntok = n_chunks_total * chunk
    info = plsc.get_sparse_core_info()
    nc, ns = info.num_cores, info.num_subcores
    nw = nc * ns
    chunks_per_w = n_chunks_total // nw
    rows_per_w = chunks_per_w * chunk

    mesh = plsc.VectorSubcoreMesh(core_axis_name="c", subcore_axis_name="s")

    @functools.partial(
        pl.kernel,
        mesh=mesh,
        out_type=jax.ShapeDtypeStruct((ntok, D), jnp.float32),
        scratch_types=[
            pltpu.VMEM((chunks_per_w, chunk), jnp.int32),
            pltpu.VMEM((chunk, D), jnp.float32),
            pltpu.VMEM((chunk, D), jnp.float32),
            pltpu.SemaphoreType.DMA,
            pltpu.SemaphoreType.DMA,
        ],
    )
    def gather_k(ids_hbm, table_hbm, out_hbm, idx_v, rows_a, rows_b, sem_a, sem_b):
        wid = lax.axis_index("s") * nc + lax.axis_index("c")
        base_chunk = wid * chunks_per_w
        base_row = wid * rows_per_w
        pltpu.sync_copy(ids_hbm.at[pl.ds(base_chunk, chunks_per_w)], idx_v)
        bufs = (rows_a, rows_b)
        sems = (sem_a, sem_b)
        copies = [None, None]
        for j in range(chunks_per_w):
            s = j % 2
            copies[s] = pltpu.async_copy(table_hbm.at[idx_v.at[j]], bufs[s], sems[s])
            if j > 0:
                copies[1 - s].wait()
                pltpu.sync_copy(bufs[1 - s],
                                out_hbm.at[pl.ds(base_row + (j - 1) * chunk, chunk)])
        last = (chunks_per_w - 1) % 2
        copies[last].wait()
        pltpu.sync_copy(bufs[last],
                        out_hbm.at[pl.ds(base_row + (chunks_per_w - 1) * chunk, chunk)])

    return gather_k(ids2d, word_table)


def _tc_add_ln(gathered, pos_table, tt_f32, type_table, gamma2d, beta2d, blk, t):
    ntok = gathered.shape[0]
    n_pos_blocks = t // blk

    def body(g_ref, pos_ref, tt_ref, ty_ref, gam_ref, bet_ref, o_ref):
        t0 = ty_ref[0:1, :]
        t1 = ty_ref[1:2, :]
        x = g_ref[...] + pos_ref[...] + t0 + tt_ref[...] * (t1 - t0)
        mean = jnp.mean(x, axis=-1, keepdims=True)
        xc = x - mean
        var = jnp.mean(xc * xc, axis=-1, keepdims=True)
        xhat = xc * lax.rsqrt(var + 1e-12)
        o_ref[...] = xhat * gam_ref[...] + bet_ref[...]

    return pl.pallas_call(
        body,
        grid=(ntok // blk,),
        in_specs=[
            pl.BlockSpec((blk, D), lambda i: (i, 0)),
            pl.BlockSpec((blk, D), lambda i: (i % n_pos_blocks, 0)),
            pl.BlockSpec((blk, 1), lambda i: (i % n_pos_blocks, 0)),
            pl.BlockSpec((2, D), lambda i: (0, 0)),
            pl.BlockSpec((1, D), lambda i: (0, 0)),
            pl.BlockSpec((1, D), lambda i: (0, 0)),
        ],
        out_specs=pl.BlockSpec((blk, D), lambda i: (i, 0)),
        out_shape=jax.ShapeDtypeStruct((ntok, D), jnp.float32),
    )(gathered, pos_table, tt_f32, type_table, gamma2d, beta2d)


def kernel(ids, token_type_ids, word_table, pos_table, type_table, ln_gamma, ln_beta):
    b, t = ids.shape
    ids2d = ids.astype(jnp.int32).reshape(-1, CHUNK)
    gathered = _sc_gather(ids2d, word_table)
    tt_f32 = token_type_ids.astype(jnp.float32).reshape(t, 1)
    out = _tc_add_ln(gathered, pos_table, tt_f32, type_table,
                     ln_gamma.reshape(1, D), ln_beta.reshape(1, D),
                     blk=1024, t=t)
    return out.reshape(b, t, D)


# speedup vs baseline: 2.4147x; 1.4440x over previous
import functools

import jax
import jax.numpy as jnp
from jax import lax
from jax.experimental import pallas as pl
from jax.experimental.pallas import tpu as pltpu
from jax.experimental.pallas import tpu_sc as plsc

D = 128
CHUNK = 128


def _sc_gather(ids2d, word_table):
    n_chunks_total, chunk = ids2d.shape
    ntok = n_chunks_total * chunk
    info = plsc.get_sparse_core_info()
    nc, ns = info.num_cores, info.num_subcores
    nw = nc * ns
    chunks_per_w = n_chunks_total // nw
    rows_per_w = chunks_per_w * chunk

    mesh = plsc.VectorSubcoreMesh(core_axis_name="c", subcore_axis_name="s")

    @functools.partial(
        pl.kernel,
        mesh=mesh,
        out_type=jax.ShapeDtypeStruct((ntok, D), jnp.float32),
        scratch_types=[
            pltpu.VMEM((chunks_per_w, chunk), jnp.int32),
            pltpu.VMEM((chunk, D), jnp.float32),
            pltpu.VMEM((chunk, D), jnp.float32),
            pltpu.SemaphoreType.DMA,
            pltpu.SemaphoreType.DMA,
        ],
    )
    def gather_k(ids_hbm, table_hbm, out_hbm, idx_v, rows_a, rows_b, sem_a, sem_b):
        wid = lax.axis_index("s") * nc + lax.axis_index("c")
        base_chunk = wid * chunks_per_w
        base_row = wid * rows_per_w
        pltpu.sync_copy(ids_hbm.at[pl.ds(base_chunk, chunks_per_w)], idx_v)
        bufs = (rows_a, rows_b)
        sems = (sem_a, sem_b)
        copies = [None, None]
        for j in range(chunks_per_w):
            s = j % 2
            copies[s] = pltpu.async_copy(table_hbm.at[idx_v.at[j]], bufs[s], sems[s])
            if j > 0:
                copies[1 - s].wait()
                pltpu.sync_copy(bufs[1 - s],
                                out_hbm.at[pl.ds(base_row + (j - 1) * chunk, chunk)])
        last = (chunks_per_w - 1) % 2
        copies[last].wait()
        pltpu.sync_copy(bufs[last],
                        out_hbm.at[pl.ds(base_row + (chunks_per_w - 1) * chunk, chunk)])

    return gather_k(ids2d, word_table)


def _tc_add_ln(gathered3d, pos_table, tt_f32, type_table, gamma2d, beta2d, blk):
    b, t, _ = gathered3d.shape

    def body(g_ref, pos_ref, tt_ref, ty_ref, gam_ref, bet_ref, o_ref):
        t0 = ty_ref[0:1, :]
        t1 = ty_ref[1:2, :]
        add = pos_ref[...] + t0 + tt_ref[...] * (t1 - t0)
        x = g_ref[...] + add[None, :, :]
        mean = jnp.mean(x, axis=-1, keepdims=True)
        xc = x - mean
        var = jnp.mean(xc * xc, axis=-1, keepdims=True)
        xhat = xc * lax.rsqrt(var + 1e-12)
        o_ref[...] = xhat * gam_ref[...] + bet_ref[...]

    return pl.pallas_call(
        body,
        grid=(t // blk,),
        in_specs=[
            pl.BlockSpec((b, blk, D), lambda i: (0, i, 0)),
            pl.BlockSpec((blk, D), lambda i: (i, 0)),
            pl.BlockSpec((blk, 1), lambda i: (i, 0)),
            pl.BlockSpec((2, D), lambda i: (0, 0)),
            pl.BlockSpec((1, D), lambda i: (0, 0)),
            pl.BlockSpec((1, D), lambda i: (0, 0)),
        ],
        out_specs=pl.BlockSpec((b, blk, D), lambda i: (0, i, 0)),
        out_shape=jax.ShapeDtypeStruct((b, t, D), jnp.float32),
    )(gathered3d, pos_table, tt_f32, type_table, gamma2d, beta2d)


def kernel(ids, token_type_ids, word_table, pos_table, type_table, ln_gamma, ln_beta):
    b, t = ids.shape
    ids2d = ids.astype(jnp.int32).reshape(-1, CHUNK)
    gathered = _sc_gather(ids2d, word_table)
    tt_f32 = token_type_ids.astype(jnp.float32).reshape(t, 1)
    out = _tc_add_ln(gathered.reshape(b, t, D), pos_table, tt_f32, type_table,
                     ln_gamma.reshape(1, D), ln_beta.reshape(1, D),
                     blk=1024)
    return out
